# baseline (speedup 1.0000x reference)
import functools

import jax
import jax.numpy as jnp
from jax import lax
from jax.experimental import pallas as pl
from jax.experimental.pallas import tpu as pltpu
from jax.experimental.pallas import tpu_sc as plsc

N = 10000
E = 320000
D = 128

NC = 2
NS = 16
NW = NC * NS
CHUNK = 128
NCHUNK = 80
NT = NCHUNK // 4
E_PER_TILE = NCHUNK * CHUNK
EPAD = NW * E_PER_TILE
NPAD = 10240
ROWS_PER_TILE = NPAD // NS


def _sc_aggregate_body(x_hbm, cv_hbm, row_hbm, zero_hbm, out_hbm,
                       row_v, cv0, cv1, cv2, cv3, rows0, rows1, acc,
                       sem_g0, sem_g1, sem_c0, sem_c1, sem_c2, sem_c3):
    c = lax.axis_index("c")
    s = lax.axis_index("s")
    wid = c * NS + s
    sem_g = (sem_g0, sem_g1)
    sem_c = (sem_c0, sem_c1, sem_c2, sem_c3)
    cvs = (cv0, cv1, cv2, cv3)
    rows = (rows0, rows1)

    pltpu.sync_copy(row_hbm.at[wid], row_v)
    pltpu.sync_copy(zero_hbm, acc.at[pl.ds(s * ROWS_PER_TILE, ROWS_PER_TILE)])

    first = pltpu.async_copy(cv_hbm.at[wid, 0], cvs[0], sem_c[0])
    for k in range(1, 4):
        pltpu.async_copy(cv_hbm.at[wid, k], cvs[k], sem_c[k])
    first.wait()
    pltpu.async_copy(x_hbm.at[cvs[0].at[0]], rows[0], sem_g[0])
    plsc.subcore_barrier()

    def scale_chunk(b, k):
        def group_body(g, carry):
            vv = lax.bitcast_convert_type(
                cvs[k][1, pl.ds(16 * g, 16)], jnp.float32)
            base = 16 * g
            for i in range(16):
                v = vv[i]
                for j in range(D // 16):
                    sl = pl.ds(16 * j, 16)
                    rows[b][base + i, sl] = rows[b][base + i, sl] * v
            return carry
        lax.fori_loop(0, CHUNK // 16, group_body, 0, unroll=False)

    def t_body(t, carry):
        for k in range(4):
            ci = 4 * t + k
            b, bn = k % 2, (k + 1) % 2
            kn = (k + 1) % 4

            def issue_next_gather():
                pltpu.make_async_copy(
                    cv_hbm.at[wid, ci + 1], cvs[kn], sem_c[kn]).wait()
                pltpu.async_copy(
                    x_hbm.at[cvs[kn].at[0]], rows[bn], sem_g[bn])

            if k == 3:
                @pl.when(t < NT - 1)
                def _():
                    issue_next_gather()
            else:
                issue_next_gather()

            pltpu.make_async_copy(
                x_hbm.at[cvs[k].at[0]], rows[b], sem_g[b]).wait()
            scale_chunk(b, k)

            @pl.when(t < NT - 1)
            def _():
                pltpu.async_copy(cv_hbm.at[wid, ci + 4], cvs[k], sem_c[k])

            pltpu.sync_copy(rows[b], acc.at[row_v.at[ci]], add=True)
        return carry
    lax.fori_loop(0, NT, t_body, 0, unroll=False)

    plsc.subcore_barrier()
    sl = pl.ds(s * ROWS_PER_TILE, ROWS_PER_TILE)
    pltpu.sync_copy(acc.at[sl], out_hbm.at[c, sl])


_sc_aggregate = functools.partial(
    pl.kernel,
    out_type=jax.ShapeDtypeStruct((NC, NPAD, D), jnp.float32),
    mesh=plsc.VectorSubcoreMesh(
        core_axis_name="c", subcore_axis_name="s", num_cores=NC,
        num_subcores=NS),
    scratch_types=[
        pltpu.VMEM((NCHUNK, CHUNK), jnp.int32),
        pltpu.VMEM((8, CHUNK), jnp.int32),
        pltpu.VMEM((8, CHUNK), jnp.int32),
        pltpu.VMEM((8, CHUNK), jnp.int32),
        pltpu.VMEM((8, CHUNK), jnp.int32),
        pltpu.VMEM((CHUNK, D), jnp.float32),
        pltpu.VMEM((CHUNK, D), jnp.float32),
        pltpu.VMEM_SHARED((NPAD, D), jnp.float32),
        pltpu.SemaphoreType.DMA,
        pltpu.SemaphoreType.DMA,
        pltpu.SemaphoreType.DMA,
        pltpu.SemaphoreType.DMA,
        pltpu.SemaphoreType.DMA,
        pltpu.SemaphoreType.DMA,
    ],
)(_sc_aggregate_body)


def _tc_matmul_body(p_ref, th_ref, b_ref, o_ref):
    mx = p_ref[0] + p_ref[1]
    o = jnp.dot(mx, th_ref[...], preferred_element_type=jnp.float32)
    o_ref[...] = jnp.maximum(o + b_ref[...], 0.0)


def _tc_matmul(partials, theta, bias):
    blk = 1000
    return pl.pallas_call(
        _tc_matmul_body,
        grid=(N // blk,),
        in_specs=[
            pl.BlockSpec((NC, blk, D), lambda i: (0, i, 0)),
            pl.BlockSpec((D, D), lambda i: (0, 0)),
            pl.BlockSpec((1, D), lambda i: (0, 0)),
        ],
        out_specs=pl.BlockSpec((blk, D), lambda i: (i, 0)),
        out_shape=jax.ShapeDtypeStruct((N, D), jnp.float32),
    )(partials, theta, bias.reshape(1, D))


def kernel(x, edge_val, theta, bias, edge_row, edge_col):
    npad = EPAD - E
    col3 = jnp.concatenate(
        [edge_col, jnp.zeros((npad,), jnp.int32)]).reshape(NW, NCHUNK, CHUNK)
    row3 = jnp.concatenate(
        [edge_row, jnp.zeros((npad,), jnp.int32)]).reshape(NW, NCHUNK, CHUNK)
    val3 = jnp.concatenate(
        [edge_val, jnp.zeros((npad,), jnp.float32)]).reshape(NW, NCHUNK, CHUNK)
    cv = jnp.concatenate(
        [col3[:, :, None, :],
         lax.bitcast_convert_type(val3, jnp.int32)[:, :, None, :],
         jnp.zeros((NW, NCHUNK, 6, CHUNK), jnp.int32)], axis=2)
    zero = jnp.zeros((ROWS_PER_TILE, D), jnp.float32)
    partials = _sc_aggregate(x, cv, row3, zero)
    return _tc_matmul(partials, theta, bias)


# speedup vs baseline: 2.4384x; 2.4384x over previous
import functools

import jax
import jax.numpy as jnp
from jax import lax
from jax.experimental import pallas as pl
from jax.experimental.pallas import tpu as pltpu
from jax.experimental.pallas import tpu_sc as plsc

N = 10000
E = 320000
D = 128

NC = 2
NS = 16
NW = NC * NS
CHUNK = 128
NCHUNK = 80
NT = NCHUNK // 4
E_PER_TILE = NCHUNK * CHUNK
EPAD = NW * E_PER_TILE
NPAD = 10240
ROWS_PER_TILE = NPAD // NS


def _sc_aggregate_body(x_hbm, cv_hbm, row_hbm, zero_hbm, out_hbm,
                       row_v, cv0, cv1, cv2, cv3, rows0, rows1, acc,
                       sem_g0, sem_g1, sem_c0, sem_c1, sem_c2, sem_c3):
    c = lax.axis_index("c")
    s = lax.axis_index("s")
    wid = c * NS + s
    sem_g = (sem_g0, sem_g1)
    sem_c = (sem_c0, sem_c1, sem_c2, sem_c3)
    cvs = (cv0, cv1, cv2, cv3)
    rows = (rows0, rows1)

    pltpu.sync_copy(row_hbm.at[wid], row_v)
    pltpu.sync_copy(zero_hbm, acc.at[pl.ds(s * ROWS_PER_TILE, ROWS_PER_TILE)])

    first = pltpu.async_copy(cv_hbm.at[wid, 0], cvs[0], sem_c[0])
    for k in range(1, 4):
        pltpu.async_copy(cv_hbm.at[wid, k], cvs[k], sem_c[k])
    first.wait()
    pltpu.async_copy(x_hbm.at[cvs[0].at[0]], rows[0], sem_g[0])
    plsc.subcore_barrier()

    def scale_chunk(b, k):
        def group_body(g, carry):
            vv = lax.bitcast_convert_type(
                cvs[k][1, pl.ds(16 * g, 16)], jnp.float32)
            base = 16 * g
            for i in range(16):
                v = vv[i]
                for j in range(D // 16):
                    sl = pl.ds(16 * j, 16)
                    rows[b][base + i, sl] = rows[b][base + i, sl] * v
            return carry
        lax.fori_loop(0, CHUNK // 16, group_body, 0, unroll=False)

    def t_body(t, carry):
        for k in range(4):
            ci = 4 * t + k
            b, bn = k % 2, (k + 1) % 2
            kn = (k + 1) % 4

            def issue_next_gather():
                pltpu.make_async_copy(
                    cv_hbm.at[wid, ci + 1], cvs[kn], sem_c[kn]).wait()
                pltpu.async_copy(
                    x_hbm.at[cvs[kn].at[0]], rows[bn], sem_g[bn])

            if k == 3:
                @pl.when(t < NT - 1)
                def _():
                    issue_next_gather()
            else:
                issue_next_gather()

            pltpu.make_async_copy(
                x_hbm.at[cvs[k].at[0]], rows[b], sem_g[b]).wait()
            scale_chunk(b, k)

            @pl.when(t < NT - 1)
            def _():
                pltpu.async_copy(cv_hbm.at[wid, ci + 4], cvs[k], sem_c[k])

            pltpu.sync_copy(rows[b], acc.at[row_v.at[ci]], add=True)
        return carry
    lax.fori_loop(0, NT, t_body, 0, unroll=False)

    plsc.subcore_barrier()
    sl = pl.ds(s * ROWS_PER_TILE, ROWS_PER_TILE)
    pltpu.sync_copy(acc.at[sl], out_hbm.at[c, sl])


_sc_aggregate = functools.partial(
    pl.kernel,
    out_type=jax.ShapeDtypeStruct((NC, NPAD, D), jnp.float32),
    mesh=plsc.VectorSubcoreMesh(
        core_axis_name="c", subcore_axis_name="s", num_cores=NC,
        num_subcores=NS),
    scratch_types=[
        pltpu.VMEM((NCHUNK, CHUNK), jnp.int32),
        pltpu.VMEM((8, CHUNK), jnp.int32),
        pltpu.VMEM((8, CHUNK), jnp.int32),
        pltpu.VMEM((8, CHUNK), jnp.int32),
        pltpu.VMEM((8, CHUNK), jnp.int32),
        pltpu.VMEM((CHUNK, D), jnp.float32),
        pltpu.VMEM((CHUNK, D), jnp.float32),
        pltpu.VMEM_SHARED((NPAD, D), jnp.float32),
        pltpu.SemaphoreType.DMA,
        pltpu.SemaphoreType.DMA,
        pltpu.SemaphoreType.DMA,
        pltpu.SemaphoreType.DMA,
        pltpu.SemaphoreType.DMA,
        pltpu.SemaphoreType.DMA,
    ],
)(_sc_aggregate_body)


def _tc_matmul_body(p_ref, th_ref, b_ref, o_ref):
    mx = p_ref[0] + p_ref[1]
    o = jnp.dot(mx, th_ref[...], preferred_element_type=jnp.float32)
    o_ref[...] = jnp.maximum(o + b_ref[...], 0.0)


def _tc_matmul(partials, theta, bias):
    blk = 1000
    return pl.pallas_call(
        _tc_matmul_body,
        grid=(N // blk,),
        in_specs=[
            pl.BlockSpec((NC, blk, D), lambda i: (0, i, 0)),
            pl.BlockSpec((D, D), lambda i: (0, 0)),
            pl.BlockSpec((1, D), lambda i: (0, 0)),
        ],
        out_specs=pl.BlockSpec((blk, D), lambda i: (i, 0)),
        out_shape=jax.ShapeDtypeStruct((N, D), jnp.float32),
    )(partials, theta, bias.reshape(1, D))


def kernel(x, edge_val, theta, bias, edge_row, edge_col):
    npad = EPAD - E
    spread = (jnp.arange(npad, dtype=jnp.int32) * 79) % N

    def prep(a, pad):
        a = jnp.concatenate([a, pad]).reshape(NCHUNK, NW, CHUNK)
        return a.transpose(1, 0, 2)

    col3 = prep(edge_col, spread)
    row3 = prep(edge_row, spread)
    val3 = prep(edge_val, jnp.zeros((npad,), jnp.float32))
    cv = jnp.concatenate(
        [col3[:, :, None, :],
         lax.bitcast_convert_type(val3, jnp.int32)[:, :, None, :],
         jnp.zeros((NW, NCHUNK, 6, CHUNK), jnp.int32)], axis=2)
    zero = jnp.zeros((ROWS_PER_TILE, D), jnp.float32)
    partials = _sc_aggregate(x, cv, row3, zero)
    return _tc_matmul(partials, theta, bias)


# speedup vs baseline: 2.4605x; 1.0091x over previous
import functools

import jax
import jax.numpy as jnp
from jax import lax
from jax.experimental import pallas as pl
from jax.experimental.pallas import tpu as pltpu
from jax.experimental.pallas import tpu_sc as plsc

N = 10000
E = 320000
D = 128

NC = 2
NS = 16
NW = NC * NS
CHUNK = 128
NCHUNK = 80
NBLK = NCHUNK // 4
NT2 = NCHUNK // 8
E_PER_TILE = NCHUNK * CHUNK
EPAD = NW * E_PER_TILE
NPAD = 10240
ROWS_PER_TILE = NPAD // NS


def _sc_aggregate_body(x_hbm, cv_hbm, row_hbm, zero_hbm, out_hbm,
                       row_v, cvb0, cvb1, rows0, rows1, acc,
                       sem_g0, sem_g1, sem_c0, sem_c1):
    c = lax.axis_index("c")
    s = lax.axis_index("s")
    wid = c * NS + s
    sem_g = (sem_g0, sem_g1)
    sem_c = (sem_c0, sem_c1)
    cvb = (cvb0, cvb1)
    rows = (rows0, rows1)

    pltpu.sync_copy(row_hbm.at[wid], row_v)
    pltpu.sync_copy(zero_hbm, acc.at[pl.ds(s * ROWS_PER_TILE, ROWS_PER_TILE)])

    first = pltpu.async_copy(cv_hbm.at[wid, 0], cvb[0], sem_c[0])
    pltpu.async_copy(cv_hbm.at[wid, 1], cvb[1], sem_c[1])
    first.wait()
    pltpu.async_copy(x_hbm.at[cvb[0].at[0]], rows[0], sem_g[0])
    plsc.subcore_barrier()

    def scale_chunk(b, blk, k):
        def group_body(g, carry):
            vv = lax.bitcast_convert_type(
                cvb[blk][4 + k, pl.ds(16 * g, 16)], jnp.float32)
            base = 16 * g
            for i in range(16):
                v = vv[i]
                for j in range(D // 16):
                    sl = pl.ds(16 * j, 16)
                    rows[b][base + i, sl] = rows[b][base + i, sl] * v
            return carry
        lax.fori_loop(0, CHUNK // 16, group_body, 0, unroll=False)

    def u_body(u, carry):
        for k in range(8):
            ci = 8 * u + k
            b, bn = k % 2, (k + 1) % 2
            kb, kr = k // 4, k % 4

            if k == 3:
                pltpu.make_async_copy(
                    cv_hbm.at[wid, 2 * u + 1], cvb[1], sem_c[1]).wait()
                pltpu.async_copy(x_hbm.at[cvb[1].at[0]], rows[bn], sem_g[bn])
            elif k == 7:
                @pl.when(u < NT2 - 1)
                def _():
                    pltpu.make_async_copy(
                        cv_hbm.at[wid, 2 * u + 2], cvb[0], sem_c[0]).wait()
                    pltpu.async_copy(
                        x_hbm.at[cvb[0].at[0]], rows[bn], sem_g[bn])
            else:
                pltpu.async_copy(
                    x_hbm.at[cvb[kb].at[kr + 1]], rows[bn], sem_g[bn])

            pltpu.make_async_copy(
                x_hbm.at[cvb[kb].at[kr]], rows[b], sem_g[b]).wait()
            scale_chunk(b, kb, kr)
            pltpu.sync_copy(rows[b], acc.at[row_v.at[ci]], add=True)

            if k == 3:
                @pl.when(u < NT2 - 1)
                def _():
                    pltpu.async_copy(cv_hbm.at[wid, 2 * u + 2], cvb[0],
                                     sem_c[0])
            elif k == 7:
                @pl.when(u < NT2 - 1)
                def _():
                    pltpu.async_copy(cv_hbm.at[wid, 2 * u + 3], cvb[1],
                                     sem_c[1])
        return carry
    lax.fori_loop(0, NT2, u_body, 0, unroll=False)

    plsc.subcore_barrier()
    sl = pl.ds(s * ROWS_PER_TILE, ROWS_PER_TILE)
    pltpu.sync_copy(acc.at[sl], out_hbm.at[c, sl])


_sc_aggregate = functools.partial(
    pl.kernel,
    out_type=jax.ShapeDtypeStruct((NC, NPAD, D), jnp.float32),
    mesh=plsc.VectorSubcoreMesh(
        core_axis_name="c", subcore_axis_name="s", num_cores=NC,
        num_subcores=NS),
    scratch_types=[
        pltpu.VMEM((NCHUNK, CHUNK), jnp.int32),
        pltpu.VMEM((8, CHUNK), jnp.int32),
        pltpu.VMEM((8, CHUNK), jnp.int32),
        pltpu.VMEM((CHUNK, D), jnp.float32),
        pltpu.VMEM((CHUNK, D), jnp.float32),
        pltpu.VMEM_SHARED((NPAD, D), jnp.float32),
        pltpu.SemaphoreType.DMA,
        pltpu.SemaphoreType.DMA,
        pltpu.SemaphoreType.DMA,
        pltpu.SemaphoreType.DMA,
    ],
)(_sc_aggregate_body)


def _tc_matmul_body(p_ref, th_ref, b_ref, o_ref):
    mx = p_ref[0] + p_ref[1]
    o = jnp.dot(mx, th_ref[...], preferred_element_type=jnp.float32)
    o_ref[...] = jnp.maximum(o + b_ref[...], 0.0)


def _tc_matmul(partials, theta, bias):
    blk = 1000
    return pl.pallas_call(
        _tc_matmul_body,
        grid=(N // blk,),
        in_specs=[
            pl.BlockSpec((NC, blk, D), lambda i: (0, i, 0)),
            pl.BlockSpec((D, D), lambda i: (0, 0)),
            pl.BlockSpec((1, D), lambda i: (0, 0)),
        ],
        out_specs=pl.BlockSpec((blk, D), lambda i: (i, 0)),
        out_shape=jax.ShapeDtypeStruct((N, D), jnp.float32),
    )(partials, theta, bias.reshape(1, D))


def kernel(x, edge_val, theta, bias, edge_row, edge_col):
    npad = EPAD - E
    spread = (jnp.arange(npad, dtype=jnp.int32) * 79) % N

    def prep(a, pad):
        a = jnp.concatenate([a, pad]).reshape(NCHUNK, NW, CHUNK)
        return a.transpose(1, 0, 2)

    col3 = prep(edge_col, spread)
    row3 = prep(edge_row, spread)
    val3 = prep(edge_val, jnp.zeros((npad,), jnp.float32))
    cv = jnp.concatenate(
        [col3.reshape(NW, NBLK, 4, CHUNK),
         lax.bitcast_convert_type(val3, jnp.int32).reshape(NW, NBLK, 4, CHUNK)],
        axis=2)
    zero = jnp.zeros((ROWS_PER_TILE, D), jnp.float32)
    partials = _sc_aggregate(x, cv, row3, zero)
    return _tc_matmul(partials, theta, bias)
